# Initial kernel scaffold; baseline (speedup 1.0000x reference)
import jax
import jax.numpy as jnp
from jax.experimental import pallas as pl


def kernel(x, values, rows, cols):
    raise NotImplementedError("write your pallas kernel here")


# speedup vs baseline: 5.3567x; 5.3567x over previous
import functools

import jax
import jax.numpy as jnp
from jax import lax
from jax.experimental import pallas as pl
from jax.experimental.pallas import tpu as pltpu
from jax.experimental.pallas import tpu_sc as plsc

IN_SIZE = 16384
OUT_SIZE = 16384
CONN = 32
BATCH = 64

NC = 2
NS = 16
NW = NC * NS
LANES = 16
BAND = OUT_SIZE // NW
CHUNK = 128
GROUPS = BATCH // LANES


def _sc_spmm(xt, vals_p, rows_p, cols_p, *, n_chunks, d_slack):
  mesh = plsc.VectorSubcoreMesh(
      core_axis_name="c", subcore_axis_name="s", num_cores=NC, num_subcores=NS)

  @functools.partial(
      pl.kernel,
      out_type=jax.ShapeDtypeStruct((OUT_SIZE * BATCH,), jnp.float32),
      mesh=mesh,
      compiler_params=pltpu.CompilerParams(
          needs_layout_passes=False, use_tc_tiling_on_sc=False),
      scratch_types=[
          pltpu.VMEM((CHUNK,), jnp.int32),
          pltpu.VMEM((CHUNK,), jnp.int32),
          pltpu.VMEM((CHUNK,), jnp.float32),
          pltpu.VMEM((CHUNK,), jnp.int32),
          pltpu.VMEM((CHUNK,), jnp.float32),
          pltpu.VMEM((CHUNK, BATCH), jnp.float32),
          pltpu.VMEM((BAND * BATCH,), jnp.float32),
          pltpu.SemaphoreType.DMA,
      ],
  )
  def kern(xt_hbm, vals_hbm, rows_hbm, cols_hbm, out_hbm,
           cidx, ridx, vraw, lrow, vv, gath, acc, gsem):
    w = lax.axis_index("c") * NS + lax.axis_index("s")
    r0 = w * BAND
    ent0 = w * (BAND * CONN)
    start = jnp.maximum(jnp.int32(0), ent0 - jnp.int32(d_slack))
    start = pl.multiple_of((start >> 3) << 3, 8)

    iotav = lax.iota(jnp.int32, LANES)
    zeros = jnp.zeros((LANES,), jnp.float32)

    def zbody(k, _):
      acc[pl.ds(k * LANES, LANES)] = zeros
      return _
    lax.fori_loop(0, (BAND * BATCH) // LANES, zbody, None)

    def chunk_body(i, _):
      off = pl.multiple_of(start + i * CHUNK, 8)
      pltpu.sync_copy(cols_hbm.at[pl.ds(off, CHUNK)], cidx)
      pltpu.sync_copy(rows_hbm.at[pl.ds(off, CHUNK)], ridx)
      pltpu.sync_copy(vals_hbm.at[pl.ds(off, CHUNK)], vraw)

      cp = pltpu.async_copy(xt_hbm.at[cidx], gath, gsem)

      for j in range(CHUNK // LANES):
        sl = pl.ds(j * LANES, LANES)
        rv = ridx[sl]
        lr = rv - r0
        ok = (lr >= 0) & (lr < BAND)
        lrow[sl] = jnp.where(ok, lr, 0) * BATCH
        vv[sl] = jnp.where(ok, vraw[sl], 0.0)

      cp.wait()

      def ent_body(j, _):
        jj = jnp.full((LANES,), j, jnp.int32)
        rsp = plsc.load_gather(lrow, [jj])
        vsp = plsc.load_gather(vv, [jj])
        for g in range(GROUPS):
          col = iotav + (g * LANES)
          vec = plsc.load_gather(gath, [jj, col])
          plsc.addupdate_scatter(acc, [rsp + col], vsp * vec)
        return _
      lax.fori_loop(0, CHUNK, ent_body, None)
      return _

    lax.fori_loop(0, n_chunks, chunk_body, None)

    pltpu.sync_copy(acc, out_hbm.at[pl.ds(r0 * BATCH, BAND * BATCH)])

  return kern(xt, vals_p, rows_p, cols_p)


def kernel(x, values, rows, cols):
  nnz = values.shape[0]
  d = OUT_SIZE * CONN - nnz
  pad_ch = -(-(d + 7) // CHUNK) * CHUNK
  if pad_ch == 0:
    pad_ch = CHUNK
  n_chunks = (BAND * CONN + pad_ch) // CHUNK
  p_len = OUT_SIZE * CONN + pad_ch

  xt = x.T.astype(jnp.float32)
  pad = p_len - nnz
  rows_p = jnp.pad(rows.astype(jnp.int32), (0, pad),
                   constant_values=jnp.int32(0x3FFFFFFF))
  cols_p = jnp.pad(cols.astype(jnp.int32), (0, pad))
  vals_p = jnp.pad(values.astype(jnp.float32), (0, pad))

  flat = _sc_spmm(xt, vals_p, rows_p, cols_p, n_chunks=n_chunks, d_slack=d)
  return flat.reshape(OUT_SIZE, BATCH).T
